# baseline (speedup 1.0000x reference)
import functools

import jax
import jax.numpy as jnp
from jax import lax
from jax.experimental import pallas as pl
from jax.experimental.pallas import tpu as pltpu
from jax.experimental.pallas import tpu_sc as plsc

B = 4096
DIM = 128
LANES = 16
NUM_WORKERS = 32
B_PER_W = B // NUM_WORKERS
CHUNKS = DIM // LANES


N_PIPE = 4
ROWS_PER_PIPE = B_PER_W // N_PIPE


def _body(cfc1_hbm, x_hbm, afc1_hbm, y_hbm, out_hbm,
          idx_x, idx_y, rows_x, rows_y, pbuf, out_v, sem_ix, sem_iy,
          *sems):
    num_cores = 2
    wid = lax.axis_index("s") * num_cores + lax.axis_index("c")
    base = wid * B_PER_W

    cpi_x = pltpu.async_copy(x_hbm.at[pl.ds(base, B_PER_W)], idx_x, sem_ix)
    cpi_y = pltpu.async_copy(y_hbm.at[pl.ds(base, B_PER_W)], idx_y, sem_iy)
    cpi_x.wait()
    cpi_y.wait()

    copies = []
    for p in range(N_PIPE):
        rows = pl.ds(p * ROWS_PER_PIPE, ROWS_PER_PIPE)
        copies.append(pltpu.async_copy(
            cfc1_hbm.at[idx_x.at[rows]], rows_x.at[rows], sems[2 * p]))
        copies.append(pltpu.async_copy(
            afc1_hbm.at[idx_y.at[rows]], rows_y.at[rows], sems[2 * p + 1]))

    lane = lax.iota(jnp.int32, LANES)

    def row(r, _):
        acc = rows_x[r, pl.ds(0, LANES)] * rows_y[r, pl.ds(0, LANES)]
        for c in range(1, CHUNKS):
            acc = acc + (rows_x[r, pl.ds(c * LANES, LANES)]
                         * rows_y[r, pl.ds(c * LANES, LANES)])
        pbuf[pl.ds(r * (LANES + 1), LANES)] = acc
        return _

    def group(g, _):
        flat = (g * LANES + lane) * (LANES + 1)
        acc = plsc.load_gather(pbuf, [flat])
        for k in range(1, LANES):
            acc = acc + plsc.load_gather(pbuf, [flat + k])
        out_v[pl.ds(g * LANES, LANES)] = acc
        return _

    for p in range(N_PIPE):
        copies[2 * p].wait()
        copies[2 * p + 1].wait()
        lax.fori_loop(p * ROWS_PER_PIPE, (p + 1) * ROWS_PER_PIPE, row, 0,
                      unroll=4)
        for g in range(p * ROWS_PER_PIPE // LANES,
                       (p + 1) * ROWS_PER_PIPE // LANES):
            group(g, 0)

    pltpu.sync_copy(out_v, out_hbm.at[pl.ds(base, B_PER_W)])


@jax.jit
def kernel(x, y, cfc1_weight, afc1_weight):
    mesh = plsc.VectorSubcoreMesh(core_axis_name="c", subcore_axis_name="s")
    run = pl.kernel(
        _body,
        out_type=jax.ShapeDtypeStruct((B,), jnp.float32),
        mesh=mesh,
        scratch_types=[
            pltpu.VMEM((B_PER_W,), jnp.int32),
            pltpu.VMEM((B_PER_W,), jnp.int32),
            pltpu.VMEM((B_PER_W, DIM), jnp.float32),
            pltpu.VMEM((B_PER_W, DIM), jnp.float32),
            pltpu.VMEM((B_PER_W * (LANES + 1),), jnp.float32),
            pltpu.VMEM((B_PER_W,), jnp.float32),
            pltpu.SemaphoreType.DMA,
            pltpu.SemaphoreType.DMA,
        ] + [pltpu.SemaphoreType.DMA] * (2 * N_PIPE),
        compiler_params=pltpu.CompilerParams(needs_layout_passes=False),
    )
    return run(cfc1_weight, x, afc1_weight, y)


# speedup vs baseline: 1.0701x; 1.0701x over previous
import functools

import jax
import jax.numpy as jnp
from jax import lax
from jax.experimental import pallas as pl
from jax.experimental.pallas import tpu as pltpu
from jax.experimental.pallas import tpu_sc as plsc

B = 4096
DIM = 128
LANES = 16
NUM_WORKERS = 32
B_PER_W = B // NUM_WORKERS
CHUNKS = DIM // LANES


N_PIPE = 4
ROWS_PER_PIPE = B_PER_W // N_PIPE


def _body(cfc1_hbm, x_hbm, afc1_hbm, y_hbm, out_hbm,
          idx_x, idx_y, rows_x, rows_y, pbuf, out_v, sem_ix, sem_iy,
          *sems):
    num_cores = 2
    wid = lax.axis_index("s") * num_cores + lax.axis_index("c")
    base = wid * B_PER_W

    cpi_x = pltpu.async_copy(x_hbm.at[pl.ds(base, B_PER_W)], idx_x, sem_ix)
    cpi_y = pltpu.async_copy(y_hbm.at[pl.ds(base, B_PER_W)], idx_y, sem_iy)
    cpi_x.wait()
    cpi_y.wait()

    copies = []
    for p in range(N_PIPE):
        rows = pl.ds(p * ROWS_PER_PIPE, ROWS_PER_PIPE)
        copies.append(pltpu.async_copy(
            cfc1_hbm.at[idx_x.at[rows]], rows_x.at[rows], sems[2 * p]))
        copies.append(pltpu.async_copy(
            afc1_hbm.at[idx_y.at[rows]], rows_y.at[rows], sems[2 * p + 1]))

    lane = lax.iota(jnp.int32, LANES)

    def row(r, _):
        acc = rows_x[r, pl.ds(0, LANES)] * rows_y[r, pl.ds(0, LANES)]
        for c in range(1, CHUNKS):
            acc = acc + (rows_x[r, pl.ds(c * LANES, LANES)]
                         * rows_y[r, pl.ds(c * LANES, LANES)])
        pbuf[pl.ds(r * (LANES + 1), LANES)] = acc
        return _

    def group(g, _):
        flat = (g * LANES + lane) * (LANES + 1)
        acc = plsc.load_gather(pbuf, [flat])
        for k in range(1, LANES):
            acc = acc + plsc.load_gather(pbuf, [flat + k])
        out_v[pl.ds(g * LANES, LANES)] = acc
        return _

    for p in range(N_PIPE):
        copies[2 * p].wait()
        copies[2 * p + 1].wait()

        @plsc.parallel_loop(p * ROWS_PER_PIPE, (p + 1) * ROWS_PER_PIPE)
        def _(r):
            row(r, 0)

        @plsc.parallel_loop(p * ROWS_PER_PIPE // LANES,
                            (p + 1) * ROWS_PER_PIPE // LANES)
        def _(g):
            group(g, 0)

    pltpu.sync_copy(out_v, out_hbm.at[pl.ds(base, B_PER_W)])


@jax.jit
def kernel(x, y, cfc1_weight, afc1_weight):
    mesh = plsc.VectorSubcoreMesh(core_axis_name="c", subcore_axis_name="s")
    run = pl.kernel(
        _body,
        out_type=jax.ShapeDtypeStruct((B,), jnp.float32),
        mesh=mesh,
        scratch_types=[
            pltpu.VMEM((B_PER_W,), jnp.int32),
            pltpu.VMEM((B_PER_W,), jnp.int32),
            pltpu.VMEM((B_PER_W, DIM), jnp.float32),
            pltpu.VMEM((B_PER_W, DIM), jnp.float32),
            pltpu.VMEM((B_PER_W * (LANES + 1),), jnp.float32),
            pltpu.VMEM((B_PER_W,), jnp.float32),
            pltpu.SemaphoreType.DMA,
            pltpu.SemaphoreType.DMA,
        ] + [pltpu.SemaphoreType.DMA] * (2 * N_PIPE),
        compiler_params=pltpu.CompilerParams(needs_layout_passes=False),
    )
    return run(cfc1_weight, x, afc1_weight, y)
